# Initial kernel scaffold; baseline (speedup 1.0000x reference)
import jax
import jax.numpy as jnp
from jax.experimental import pallas as pl


def kernel(inputs, token_table, pos_table):
    raise NotImplementedError("write your pallas kernel here")


# speedup vs baseline: 3.2849x; 3.2849x over previous
import functools

import jax
import jax.numpy as jnp
from jax import lax
from jax.experimental import pallas as pl
from jax.experimental.pallas import tpu as pltpu
from jax.experimental.pallas import tpu_sc as plsc

BATCH = 4096
SEQ = 200
EMBED = 64
NC, NS, LANES = 2, 16, 16
NW = NC * NS
ROWS_PER_W = BATCH // NW
HALF = SEQ // 2


def _body(idx_hbm, tok_hbm, pos_hbm, out_hbm, idx_v, pos_v, rows_v, sem):
    wid = lax.axis_index("s") * NC + lax.axis_index("c")

    pltpu.sync_copy(idx_hbm.at[pl.ds(wid * (2 * ROWS_PER_W), 2 * ROWS_PER_W)],
                    idx_v)
    pltpu.sync_copy(pos_hbm, pos_v)

    def row(r, _):
        c1 = pltpu.async_copy(tok_hbm.at[idx_v.at[2 * r]],
                              rows_v.at[pl.ds(0, HALF)], sem)
        c2 = pltpu.async_copy(tok_hbm.at[idx_v.at[2 * r + 1]],
                              rows_v.at[pl.ds(HALF, HALF)], sem)
        c1.wait()
        c2.wait()

        def add_pos(s, _):
            for d in range(EMBED // LANES):
                x = pos_v[s, pl.ds(d * LANES, LANES)]
                plsc.addupdate(rows_v.at[s, pl.ds(d * LANES, LANES)], x)
            return ()

        lax.fori_loop(0, SEQ, add_pos, (), unroll=2)
        pltpu.sync_copy(rows_v,
                        out_hbm.at[pl.ds((wid * ROWS_PER_W + r) * SEQ, SEQ)])
        return ()

    lax.fori_loop(0, ROWS_PER_W, row, ())


@functools.partial(
    pl.kernel,
    out_type=jax.ShapeDtypeStruct((BATCH * SEQ, EMBED), jnp.float32),
    mesh=plsc.VectorSubcoreMesh(core_axis_name="c", subcore_axis_name="s",
                                num_cores=NC, num_subcores=NS),
    scratch_types=[
        pltpu.VMEM((2 * ROWS_PER_W, HALF), jnp.int32),
        pltpu.VMEM((SEQ, EMBED), jnp.float32),
        pltpu.VMEM((SEQ, EMBED), jnp.float32),
        pltpu.SemaphoreType.DMA,
    ],
    compiler_params=pltpu.CompilerParams(use_tc_tiling_on_sc=False),
)
def _embed_kernel(idx_hbm, tok_hbm, pos_hbm, out_hbm, idx_v, pos_v, rows_v,
                  sem):
    _body(idx_hbm, tok_hbm, pos_hbm, out_hbm, idx_v, pos_v, rows_v, sem)


def kernel(inputs, token_table, pos_table):
    idx = inputs.astype(jnp.int32).reshape(BATCH * SEQ // HALF, HALF)
    out = _embed_kernel(idx, token_table, pos_table)
    return out.reshape(BATCH, SEQ, EMBED)


# speedup vs baseline: 4.2333x; 1.2887x over previous
import functools

import jax
import jax.numpy as jnp
from jax import lax
from jax.experimental import pallas as pl
from jax.experimental.pallas import tpu as pltpu
from jax.experimental.pallas import tpu_sc as plsc

BATCH = 4096
SEQ = 200
EMBED = 64
NC, NS, LANES = 2, 16, 16
NW = NC * NS
ROWS_PER_W = BATCH // NW
HALF = SEQ // 2
NBUF = 4
DELTA = 2


def _body(idx_hbm, tok_hbm, pos_hbm, out_hbm, idx_v, pos_v, bufs, gsems,
          ssems):
    wid = lax.axis_index("s") * NC + lax.axis_index("c")

    pltpu.sync_copy(idx_hbm.at[pl.ds(wid * (2 * ROWS_PER_W), 2 * ROWS_PER_W)],
                    idx_v)
    pltpu.sync_copy(pos_hbm, pos_v)

    def fire_gather(b, r):
        pltpu.async_copy(tok_hbm.at[idx_v.at[2 * r]],
                         bufs[b].at[pl.ds(0, HALF)], gsems[b])
        pltpu.async_copy(tok_hbm.at[idx_v.at[2 * r + 1]],
                         bufs[b].at[pl.ds(HALF, HALF)], gsems[b])

    def wait_gather(b):
        for _ in range(2):
            pltpu.make_async_copy(tok_hbm.at[idx_v.at[0]],
                                  bufs[b].at[pl.ds(0, HALF)],
                                  gsems[b]).wait()

    def fire_store(b, r):
        pltpu.async_copy(
            bufs[b], out_hbm.at[pl.ds((wid * ROWS_PER_W + r) * SEQ, SEQ)],
            ssems[b])

    def wait_store(b):
        pltpu.make_async_copy(bufs[b], out_hbm.at[pl.ds(0, SEQ)],
                              ssems[b]).wait()

    def add_pos(b):
        buf = bufs[b]

        @plsc.parallel_loop(0, SEQ, 1, unroll=4)
        def _(s):
            for d in range(EMBED // LANES):
                sl = pl.ds(d * LANES, LANES)
                plsc.addupdate(buf.at[s, sl], pos_v[s, sl])

    for r in range(DELTA):
        fire_gather(r, r)

    def outer(g, _):
        r0 = g * NBUF
        for b in range(NBUF):
            r = r0 + b
            rf = r + DELTA
            bf = (b + DELTA) % NBUF

            @pl.when(jnp.logical_and(rf >= NBUF, rf < ROWS_PER_W))
            def _():
                wait_store(bf)

            @pl.when(rf < ROWS_PER_W)
            def _():
                fire_gather(bf, rf)

            wait_gather(b)
            add_pos(b)
            fire_store(b, r)
        return ()

    lax.fori_loop(0, ROWS_PER_W // NBUF, outer, ())
    for b in range(NBUF):
        wait_store(b)


@functools.partial(
    pl.kernel,
    out_type=jax.ShapeDtypeStruct((BATCH * SEQ, EMBED), jnp.float32),
    mesh=plsc.VectorSubcoreMesh(core_axis_name="c", subcore_axis_name="s",
                                num_cores=NC, num_subcores=NS),
    scratch_types=[
        pltpu.VMEM((2 * ROWS_PER_W, HALF), jnp.int32),
        pltpu.VMEM((SEQ, EMBED), jnp.float32),
    ] + [pltpu.VMEM((SEQ, EMBED), jnp.float32)] * NBUF
      + [pltpu.SemaphoreType.DMA] * (2 * NBUF),
    compiler_params=pltpu.CompilerParams(use_tc_tiling_on_sc=False),
)
def _embed_kernel(idx_hbm, tok_hbm, pos_hbm, out_hbm, idx_v, pos_v, *rest):
    bufs = rest[:NBUF]
    gsems = rest[NBUF:2 * NBUF]
    ssems = rest[2 * NBUF:]
    _body(idx_hbm, tok_hbm, pos_hbm, out_hbm, idx_v, pos_v, bufs, gsems,
          ssems)


def kernel(inputs, token_table, pos_table):
    idx = inputs.astype(jnp.int32).reshape(BATCH * SEQ // HALF, HALF)
    out = _embed_kernel(idx, token_table, pos_table)
    return out.reshape(BATCH, SEQ, EMBED)


# speedup vs baseline: 7.3833x; 1.7441x over previous
import functools

import jax
import jax.numpy as jnp
from jax import lax
from jax.experimental import pallas as pl
from jax.experimental.pallas import tpu as pltpu
from jax.experimental.pallas import tpu_sc as plsc

BATCH = 4096
SEQ = 200
EMBED = 64
NC, NS, LANES = 2, 16, 16
NW = NC * NS
ROWS_PER_W = BATCH // NW
HALF = SEQ // 2
NBUF = 4
DELTA = 2


def _body(idx_hbm, tok_hbm, pos_hbm, out_hbm, idx_v, pos_v, bufs, gsems,
          ssems):
    wid = lax.axis_index("s") * NC + lax.axis_index("c")

    pltpu.sync_copy(idx_hbm.at[pl.ds(wid * (2 * ROWS_PER_W), 2 * ROWS_PER_W)],
                    idx_v)
    pltpu.sync_copy(pos_hbm, pos_v)

    def fire_gather(b, r):
        pltpu.async_copy(tok_hbm.at[idx_v.at[2 * r]],
                         bufs[b].at[pl.ds(0, HALF)], gsems[b])
        pltpu.async_copy(tok_hbm.at[idx_v.at[2 * r + 1]],
                         bufs[b].at[pl.ds(HALF, HALF)], gsems[b])

    def wait_gather(b):
        for _ in range(2):
            pltpu.make_async_copy(tok_hbm.at[idx_v.at[0]],
                                  bufs[b].at[pl.ds(0, HALF)],
                                  gsems[b]).wait()

    def fire_store(b, r):
        pltpu.async_copy(
            bufs[b],
            out_hbm.at[pl.ds((wid * ROWS_PER_W + r) * SEQ, SEQ),
                       pl.ds(0, EMBED)],
            ssems[b])

    def wait_store(b):
        pltpu.make_async_copy(bufs[b],
                              out_hbm.at[pl.ds(0, SEQ), pl.ds(0, EMBED)],
                              ssems[b]).wait()

    def add_pos(b):
        buf = bufs[b]

        @plsc.parallel_loop(0, SEQ, 1, unroll=4)
        def _(s):
            for d in range(EMBED // LANES):
                sl = pl.ds(d * LANES, LANES)
                plsc.addupdate(buf.at[s, sl], pos_v[s, sl])

    for r in range(DELTA):
        fire_gather(r, r)

    def outer(g, _):
        r0 = g * NBUF
        for b in range(NBUF):
            r = r0 + b
            rf = r + DELTA
            bf = (b + DELTA) % NBUF

            @pl.when(jnp.logical_and(rf >= NBUF, rf < ROWS_PER_W))
            def _():
                wait_store(bf)

            @pl.when(rf < ROWS_PER_W)
            def _():
                fire_gather(bf, rf)

            wait_gather(b)
            add_pos(b)
            fire_store(b, r)
        return ()

    lax.fori_loop(0, ROWS_PER_W // NBUF, outer, ())
    for b in range(NBUF):
        wait_store(b)


@functools.partial(
    pl.kernel,
    out_type=jax.ShapeDtypeStruct((BATCH * SEQ, 2 * EMBED), jnp.float32),
    mesh=plsc.VectorSubcoreMesh(core_axis_name="c", subcore_axis_name="s",
                                num_cores=NC, num_subcores=NS),
    scratch_types=[
        pltpu.VMEM((2 * ROWS_PER_W, HALF), jnp.int32),
        pltpu.VMEM((SEQ, EMBED), jnp.float32),
    ] + [pltpu.VMEM((SEQ, EMBED), jnp.float32)] * NBUF
      + [pltpu.SemaphoreType.DMA] * (2 * NBUF),
    compiler_params=pltpu.CompilerParams(use_tc_tiling_on_sc=False),
)
def _embed_kernel(idx_hbm, tok_hbm, pos_hbm, out_hbm, idx_v, pos_v, *rest):
    bufs = rest[:NBUF]
    gsems = rest[NBUF:2 * NBUF]
    ssems = rest[2 * NBUF:]
    _body(idx_hbm, tok_hbm, pos_hbm, out_hbm, idx_v, pos_v, bufs, gsems,
          ssems)


def kernel(inputs, token_table, pos_table):
    idx = inputs.astype(jnp.int32).reshape(BATCH * SEQ // HALF, HALF)
    out = _embed_kernel(idx, token_table, pos_table)
    return out[:, :EMBED].reshape(BATCH, SEQ, EMBED)


# speedup vs baseline: 7.3982x; 1.0020x over previous
import functools

import jax
import jax.numpy as jnp
from jax import lax
from jax.experimental import pallas as pl
from jax.experimental.pallas import tpu as pltpu
from jax.experimental.pallas import tpu_sc as plsc

BATCH = 4096
SEQ = 200
EMBED = 64
NC, NS, LANES = 2, 16, 16
NW = NC * NS
ROWS_PER_W = BATCH // NW
HALF = SEQ // 2
NBUF = 4
DELTA = 2


def _body(idx_hbm, tok_hbm, pos_hbm, out_hbm, idx_v, pos_v, bufs, gsems,
          ssems):
    wid = lax.axis_index("s") * NC + lax.axis_index("c")

    pltpu.sync_copy(idx_hbm.at[pl.ds(wid * (2 * ROWS_PER_W), 2 * ROWS_PER_W)],
                    idx_v)
    pltpu.sync_copy(pos_hbm, pos_v)

    def fire_gather(b, r):
        pltpu.async_copy(tok_hbm.at[idx_v.at[2 * r]],
                         bufs[b].at[pl.ds(0, HALF)], gsems[b])
        pltpu.async_copy(tok_hbm.at[idx_v.at[2 * r + 1]],
                         bufs[b].at[pl.ds(HALF, HALF)], gsems[b])

    def wait_gather(b):
        for _ in range(2):
            pltpu.make_async_copy(tok_hbm.at[idx_v.at[0]],
                                  bufs[b].at[pl.ds(0, HALF)],
                                  gsems[b]).wait()

    def fire_store(b, r):
        pltpu.async_copy(
            bufs[b],
            out_hbm.at[pl.ds((wid * ROWS_PER_W + r) * SEQ, SEQ),
                       pl.ds(0, EMBED)],
            ssems[b])

    def wait_store(b):
        pltpu.make_async_copy(bufs[b],
                              out_hbm.at[pl.ds(0, SEQ), pl.ds(0, EMBED)],
                              ssems[b]).wait()

    def add_pos(b):
        buf = bufs[b]

        @plsc.parallel_loop(0, SEQ, 1, unroll=4)
        def _(s):
            for d in range(EMBED // LANES):
                sl = pl.ds(d * LANES, LANES)
                plsc.addupdate(buf.at[s, sl], pos_v[s, sl])

    for r in range(DELTA):
        fire_gather(r, r)

    def outer(g, _):
        r0 = g * NBUF
        for b in range(NBUF):
            r = r0 + b
            rf = r + DELTA
            bf = (b + DELTA) % NBUF

            @pl.when(jnp.logical_and(rf >= NBUF, rf < ROWS_PER_W))
            def _():
                wait_store(bf)

            @pl.when(rf < ROWS_PER_W)
            def _():
                fire_gather(bf, rf)

            wait_gather(b)
            add_pos(b)
            fire_store(b, r)
        return ()

    lax.fori_loop(0, ROWS_PER_W // NBUF, outer, ())
    for b in range(NBUF):
        wait_store(b)


@functools.partial(
    pl.kernel,
    out_type=jax.ShapeDtypeStruct((BATCH * SEQ, 2 * EMBED), jnp.float32),
    mesh=plsc.VectorSubcoreMesh(core_axis_name="c", subcore_axis_name="s",
                                num_cores=NC, num_subcores=NS),
    scratch_types=[
        pltpu.VMEM((2 * ROWS_PER_W, HALF), jnp.int32),
        pltpu.VMEM((SEQ, EMBED), jnp.float32),
    ] + [pltpu.VMEM((SEQ, EMBED), jnp.float32)] * NBUF
      + [pltpu.SemaphoreType.DMA] * (2 * NBUF),
    compiler_params=pltpu.CompilerParams(use_tc_tiling_on_sc=False),
)
def _embed_kernel(idx_hbm, tok_hbm, pos_hbm, out_hbm, idx_v, pos_v, *rest):
    bufs = rest[:NBUF]
    gsems = rest[NBUF:2 * NBUF]
    ssems = rest[2 * NBUF:]
    _body(idx_hbm, tok_hbm, pos_hbm, out_hbm, idx_v, pos_v, bufs, gsems,
          ssems)


def kernel(inputs, token_table, pos_table):
    idx = inputs.astype(jnp.int32).reshape(BATCH * SEQ // HALF, HALF)
    out = _embed_kernel(idx, token_table, pos_table)
    return out.reshape(BATCH, SEQ, 2 * EMBED)[:, :, :EMBED]
